# baseline (speedup 1.0000x reference)
import numpy as np

import jax
import jax.numpy as jnp
from jax import lax
from jax.experimental import pallas as pl
from jax.experimental.pallas import tpu as pltpu
from jax.experimental.pallas import tpu_sc as plsc

B = 16384
H = 128
NC = 2
NS = 16
NW = NC * NS
BPW = B // NW
CHUNK = 128
NCHUNK = BPW // CHUNK
NVEC = H // 16

BM = 2048

_PERM = np.arange(H, dtype=np.int32)


def _gather_body(pt_hbm, ct_hbm, bt_hbm, ip_hbm, ic_hbm, ib_hbm,
                 out_hbm,
                 idxp, idxc, idxb, acc, buf, ob, isem, psem, bsem0, bsem1,
                 osem):
    wid = lax.axis_index("s") * NC + lax.axis_index("c")
    base_row = wid * NCHUNK
    base = wid * BPW

    idescs = [
        pltpu.async_copy(ip_hbm.at[pl.ds(base_row, NCHUNK)], idxp, isem),
        pltpu.async_copy(ic_hbm.at[pl.ds(base_row, NCHUNK)], idxc, isem),
        pltpu.async_copy(ib_hbm.at[pl.ds(base_row, NCHUNK)], idxb, isem),
    ]
    for d in idescs:
        d.wait()

    pdescs = [
        pltpu.async_copy(pt_hbm.at[idxp.at[j]],
                         acc.at[pl.ds(j * CHUNK, CHUNK)], psem)
        for j in range(NCHUNK)
    ]

    segs = []
    for j in range(NCHUNK):
        segs.append((ct_hbm, idxc, j))
        segs.append((bt_hbm, idxb, j))
    bsems = (bsem0, bsem1)

    def fire(s):
        tab, idx, j = segs[s]
        k = s % 2
        return pltpu.async_copy(tab.at[idx.at[j]], buf.at[k], bsems[k])

    descs = {0: fire(0), 1: fire(1)}
    for d in pdescs:
        d.wait()

    wdescs = {}
    for s in range(len(segs)):
        k = s % 2
        descs.pop(s).wait()
        cbase = segs[s][2] * CHUNK

        def add_body(i, _, k=k, cbase=cbase):
            r0 = i * 2
            r1 = r0 + 1
            for u in range(NVEC):
                c = u * 16
                plsc.addupdate(acc.at[cbase + r0, pl.ds(c, 16)],
                               buf[k, r0, pl.ds(c, 16)])
                plsc.addupdate(acc.at[cbase + r1, pl.ds(c, 16)],
                               buf[k, r1, pl.ds(c, 16)])
            return _

        lax.fori_loop(0, CHUNK // 2, add_body, None)
        if s + 2 < len(segs):
            descs[s + 2] = fire(s + 2)

        if s % 2 == 1:
            j = s // 2
            ko = j % 2
            if j - 2 in wdescs:
                wdescs.pop(j - 2).wait()

            def pack_body(i, _, ko=ko, cbase=cbase):
                r = i
                for q in range(H // 32):
                    a = acc[cbase + r, pl.ds(32 * q, 16)]
                    b = acc[cbase + r, pl.ds(32 * q + 16, 16)]
                    ob[ko, r, pl.ds(32 * q, 32)] = plsc.pack(
                        a, b, format=plsc.PackFormat.INTERLEAVED)
                return _

            lax.fori_loop(0, CHUNK, pack_body, None)
            wdescs[j] = pltpu.async_copy(
                ob.at[ko], out_hbm.at[pl.ds(base + j * CHUNK, CHUNK)], osem)

    for d in wdescs.values():
        d.wait()


_gather = pl.kernel(
    _gather_body,
    out_type=jax.ShapeDtypeStruct((B, H), jnp.bfloat16),
    mesh=plsc.VectorSubcoreMesh(core_axis_name="c", subcore_axis_name="s",
                                num_cores=NC, num_subcores=NS),
    compiler_params=pltpu.CompilerParams(needs_layout_passes=False),
    scratch_types=[
        pltpu.VMEM((NCHUNK, CHUNK), jnp.int32),
        pltpu.VMEM((NCHUNK, CHUNK), jnp.int32),
        pltpu.VMEM((NCHUNK, CHUNK), jnp.int32),
        pltpu.VMEM((BPW, H), jnp.float32),
        pltpu.VMEM((2, CHUNK, H), jnp.float32),
        pltpu.VMEM((2, CHUNK, H), jnp.bfloat16),
        pltpu.SemaphoreType.DMA,
        pltpu.SemaphoreType.DMA,
        pltpu.SemaphoreType.DMA,
        pltpu.SemaphoreType.DMA,
        pltpu.SemaphoreType.DMA,
    ],
)


def _mlp_body(h_ref, w1_ref, b1_ref, w2_ref, b2_ref, out_ref):
    h = h_ref[...]
    w1 = w1_ref[...].astype(jnp.bfloat16)
    w2 = w2_ref[...].astype(jnp.bfloat16)
    a = jnp.dot(h, w1, preferred_element_type=jnp.float32) + b1_ref[...]
    a = a * jax.nn.sigmoid(a)
    out_ref[...] = (jnp.dot(a.astype(jnp.bfloat16), w2,
                            preferred_element_type=jnp.float32) + b2_ref[...])


def _mlp(hidden, W1p, b1, W2, b2):
    grid = (B // BM,)
    row_spec = pl.BlockSpec((BM, H), lambda i: (i, 0))
    full = pl.BlockSpec((H, H), lambda i: (0, 0))
    bias = pl.BlockSpec((1, H), lambda i: (0, 0))
    return pl.pallas_call(
        _mlp_body,
        grid=grid,
        in_specs=[row_spec, full, bias, full, bias],
        out_specs=row_spec,
        out_shape=jax.ShapeDtypeStruct((B, H), jnp.float32),
    )(hidden, W1p, b1.reshape(1, H), W2, b2.reshape(1, H))


def kernel(perturbation, cell_type, batch, perturb_table, cell_table,
           batch_table, W1, b1, W2, b2):
    ip = perturbation.astype(jnp.int32).reshape(B // CHUNK, CHUNK)
    ic = cell_type.astype(jnp.int32).reshape(B // CHUNK, CHUNK)
    ib = batch.astype(jnp.int32).reshape(B // CHUNK, CHUNK)
    hidden = _gather(perturb_table, cell_table, batch_table, ip, ic, ib)
    W1p = W1[jnp.asarray(_PERM)]
    return _mlp(hidden, W1p, b1, W2, b2)


# speedup vs baseline: 1.0200x; 1.0200x over previous
import numpy as np

import jax
import jax.numpy as jnp
from jax import lax
from jax.experimental import pallas as pl
from jax.experimental.pallas import tpu as pltpu
from jax.experimental.pallas import tpu_sc as plsc

B = 16384
H = 128
NC = 2
NS = 16
NW = NC * NS
CHUNK = 128
NVEC = H // 16

HALF = B // 2
BPW = HALF // NW
NCHUNK = BPW // CHUNK

BM = 2048
NBLK = HALF // BM


def _make_gather_body(half):
    row0 = half * HALF

    def _gather_body(pt_hbm, ct_hbm, bt_hbm, ip_hbm, ic_hbm, ib_hbm,
                     out_hbm,
                     idxp, idxc, idxb, acc, buf, isem, psem, bsem0, bsem1):
        wid = lax.axis_index("s") * NC + lax.axis_index("c")
        base_row = row0 // CHUNK + wid * NCHUNK
        base = wid * BPW

        idescs = [
            pltpu.async_copy(ip_hbm.at[pl.ds(base_row, NCHUNK)], idxp, isem),
            pltpu.async_copy(ic_hbm.at[pl.ds(base_row, NCHUNK)], idxc, isem),
            pltpu.async_copy(ib_hbm.at[pl.ds(base_row, NCHUNK)], idxb, isem),
        ]
        for d in idescs:
            d.wait()

        pdescs = [
            pltpu.async_copy(pt_hbm.at[idxp.at[j]],
                             acc.at[pl.ds(j * CHUNK, CHUNK)], psem)
            for j in range(NCHUNK)
        ]

        segs = ([(ct_hbm, idxc, j) for j in range(NCHUNK)]
                + [(bt_hbm, idxb, j) for j in range(NCHUNK)])
        bsems = (bsem0, bsem1)

        def fire(s):
            tab, idx, j = segs[s]
            k = s % 2
            return pltpu.async_copy(tab.at[idx.at[j]], buf.at[k], bsems[k])

        descs = {0: fire(0), 1: fire(1)}
        for d in pdescs:
            d.wait()

        for s in range(len(segs)):
            k = s % 2
            descs.pop(s).wait()
            cbase = segs[s][2] * CHUNK

            def add_body(i, _, k=k, cbase=cbase):
                r0 = i * 2
                r1 = r0 + 1
                for u in range(NVEC):
                    c = u * 16
                    plsc.addupdate(acc.at[cbase + r0, pl.ds(c, 16)],
                                   buf[k, r0, pl.ds(c, 16)])
                    plsc.addupdate(acc.at[cbase + r1, pl.ds(c, 16)],
                                   buf[k, r1, pl.ds(c, 16)])
                return _

            lax.fori_loop(0, CHUNK // 2, add_body, None)
            if s + 2 < len(segs):
                descs[s + 2] = fire(s + 2)

        pltpu.sync_copy(acc, out_hbm.at[pl.ds(base, BPW)])

    return _gather_body


def _make_gather(half):
    return pl.kernel(
        _make_gather_body(half),
        out_type=jax.ShapeDtypeStruct((HALF, H), jnp.float32),
        mesh=plsc.VectorSubcoreMesh(core_axis_name="c", subcore_axis_name="s",
                                    num_cores=NC, num_subcores=NS),
        scratch_types=[
            pltpu.VMEM((NCHUNK, CHUNK), jnp.int32),
            pltpu.VMEM((NCHUNK, CHUNK), jnp.int32),
            pltpu.VMEM((NCHUNK, CHUNK), jnp.int32),
            pltpu.VMEM((BPW, H), jnp.float32),
            pltpu.VMEM((2, CHUNK, H), jnp.float32),
            pltpu.SemaphoreType.DMA,
            pltpu.SemaphoreType.DMA,
            pltpu.SemaphoreType.DMA,
            pltpu.SemaphoreType.DMA,
        ],
    )


_gather0 = _make_gather(0)
_gather1 = _make_gather(1)


def _mlp_body0(h_ref, w1_ref, b1_ref, w2_ref, b2_ref, out_ref):
    h = h_ref[...]
    a = jnp.dot(h, w1_ref[...], preferred_element_type=jnp.float32) + b1_ref[...]
    a = a * jax.nn.sigmoid(a)
    out_ref[...] = (jnp.dot(a, w2_ref[...], preferred_element_type=jnp.float32)
                    + b2_ref[...])


def _mlp_body1(h_ref, prev_ref, w1_ref, b1_ref, w2_ref, b2_ref, out_ref):
    del prev_ref
    _mlp_body0(h_ref, w1_ref, b1_ref, w2_ref, b2_ref, out_ref)


_row_spec = pl.BlockSpec((BM, H), lambda i: (i, 0))
_full = pl.BlockSpec((H, H), lambda i: (0, 0))
_bias = pl.BlockSpec((1, H), lambda i: (0, 0))
_any = pl.BlockSpec(memory_space=pl.ANY)


def _mlp0(h0, W1, b1, W2, b2):
    return pl.pallas_call(
        _mlp_body0,
        grid=(NBLK,),
        in_specs=[_row_spec, _full, _bias, _full, _bias],
        out_specs=pl.BlockSpec((BM, H), lambda i: (i, 0)),
        out_shape=jax.ShapeDtypeStruct((B, H), jnp.float32),
    )(h0, W1, b1, W2, b2)


def _mlp1(h1, prev, W1, b1, W2, b2):
    return pl.pallas_call(
        _mlp_body1,
        grid=(NBLK,),
        in_specs=[_row_spec, _any, _full, _bias, _full, _bias],
        out_specs=pl.BlockSpec((BM, H), lambda i: (i + NBLK, 0)),
        out_shape=jax.ShapeDtypeStruct((B, H), jnp.float32),
        input_output_aliases={1: 0},
    )(h1, prev, W1, b1, W2, b2)


def kernel(perturbation, cell_type, batch, perturb_table, cell_table,
           batch_table, W1, b1, W2, b2):
    ip = perturbation.astype(jnp.int32).reshape(B // CHUNK, CHUNK)
    ic = cell_type.astype(jnp.int32).reshape(B // CHUNK, CHUNK)
    ib = batch.astype(jnp.int32).reshape(B // CHUNK, CHUNK)
    b1r = b1.reshape(1, H)
    b2r = b2.reshape(1, H)
    h0 = _gather0(perturb_table, cell_table, batch_table, ip, ic, ib)
    h1 = _gather1(perturb_table, cell_table, batch_table, ip, ic, ib)
    out0 = _mlp0(h0, W1, b1r, W2, b2r)
    return _mlp1(h1, out0, W1, b1r, W2, b2r)


# speedup vs baseline: 1.1324x; 1.1102x over previous
import jax
import jax.numpy as jnp
from jax import lax
from jax.experimental import pallas as pl
from jax.experimental.pallas import tpu as pltpu
from jax.experimental.pallas import tpu_sc as plsc

B = 16384
H = 128
NC = 2
NS = 16
NW = NC * NS
BPW = B // NW
CHUNK = 128
NCHUNK = BPW // CHUNK
NVEC = H // 16

BM = 2048


def _gather_body(pt_hbm, ct_hbm, bt_hbm, ip_hbm, ic_hbm, ib_hbm,
                 out_hbm,
                 idxp, idxc, idxb, acc, buf, isem, psem, bsem0, bsem1, osem):
    wid = lax.axis_index("s") * NC + lax.axis_index("c")
    base = wid * BPW

    idescs = []
    for src, dst in ((ip_hbm, idxp), (ic_hbm, idxc), (ib_hbm, idxb)):
        for j in range(NCHUNK):
            idescs.append(pltpu.async_copy(
                src.at[pl.ds(base + j * CHUNK, CHUNK)], dst.at[j], isem))
    for d in idescs:
        d.wait()

    pdescs = [
        pltpu.async_copy(pt_hbm.at[idxp.at[j]],
                         acc.at[pl.ds(j * CHUNK, CHUNK)], psem)
        for j in range(NCHUNK)
    ]

    segs = []
    for j in range(NCHUNK):
        segs.append((ct_hbm, idxc, j))
        segs.append((bt_hbm, idxb, j))
    bsems = (bsem0, bsem1)

    def fire(s):
        tab, idx, j = segs[s]
        k = s % 2
        return pltpu.async_copy(tab.at[idx.at[j]], buf.at[k], bsems[k])

    descs = {0: fire(0), 1: fire(1)}
    wdescs = []
    for s in range(len(segs)):
        k = s % 2
        j = segs[s][2]
        if s % 2 == 0:
            pdescs[j].wait()
        descs.pop(s).wait()
        cbase = j * CHUNK

        def add_body(i, _, k=k, cbase=cbase):
            r0 = i * 2
            r1 = r0 + 1
            for u in range(NVEC):
                c = u * 16
                plsc.addupdate(acc.at[cbase + r0, pl.ds(c, 16)],
                               buf[k, r0, pl.ds(c, 16)])
                plsc.addupdate(acc.at[cbase + r1, pl.ds(c, 16)],
                               buf[k, r1, pl.ds(c, 16)])
            return _

        lax.fori_loop(0, CHUNK // 2, add_body, None)
        if s + 2 < len(segs):
            descs[s + 2] = fire(s + 2)
        if s % 2 == 1:
            wdescs.append(pltpu.async_copy(
                acc.at[pl.ds(cbase, CHUNK)],
                out_hbm.at[pl.ds(base + cbase, CHUNK)], osem))

    for d in wdescs:
        d.wait()


_gather = pl.kernel(
    _gather_body,
    out_type=jax.ShapeDtypeStruct((B, H), jnp.float32),
    mesh=plsc.VectorSubcoreMesh(core_axis_name="c", subcore_axis_name="s",
                                num_cores=NC, num_subcores=NS),
    scratch_types=[
        pltpu.VMEM((NCHUNK, CHUNK), jnp.int32),
        pltpu.VMEM((NCHUNK, CHUNK), jnp.int32),
        pltpu.VMEM((NCHUNK, CHUNK), jnp.int32),
        pltpu.VMEM((BPW, H), jnp.float32),
        pltpu.VMEM((2, CHUNK, H), jnp.float32),
        pltpu.SemaphoreType.DMA,
        pltpu.SemaphoreType.DMA,
        pltpu.SemaphoreType.DMA,
        pltpu.SemaphoreType.DMA,
        pltpu.SemaphoreType.DMA,
    ],
)


def _mlp_body(h_ref, w1_ref, b1_ref, w2_ref, b2_ref, out_ref):
    h = h_ref[...]
    a = jnp.dot(h, w1_ref[...], preferred_element_type=jnp.float32) + b1_ref[...]
    a = a * jax.nn.sigmoid(a)
    out_ref[...] = (jnp.dot(a, w2_ref[...], preferred_element_type=jnp.float32)
                    + b2_ref[...])


def _mlp(hidden, W1, b1, W2, b2):
    grid = (B // BM,)
    row_spec = pl.BlockSpec((BM, H), lambda i: (i, 0))
    full = pl.BlockSpec((H, H), lambda i: (0, 0))
    bias = pl.BlockSpec((1, H), lambda i: (0, 0))
    return pl.pallas_call(
        _mlp_body,
        grid=grid,
        in_specs=[row_spec, full, bias, full, bias],
        out_specs=row_spec,
        out_shape=jax.ShapeDtypeStruct((B, H), jnp.float32),
    )(hidden, W1, b1.reshape(1, H), W2, b2.reshape(1, H))


def kernel(perturbation, cell_type, batch, perturb_table, cell_table,
           batch_table, W1, b1, W2, b2):
    ip = perturbation.astype(jnp.int32)
    ic = cell_type.astype(jnp.int32)
    ib = batch.astype(jnp.int32)
    hidden = _gather(perturb_table, cell_table, batch_table, ip, ic, ib)
    return _mlp(hidden, W1, b1, W2, b2)


# speedup vs baseline: 1.1862x; 1.0475x over previous
import jax
import jax.numpy as jnp
from jax import lax
from jax.experimental import pallas as pl
from jax.experimental.pallas import tpu as pltpu
from jax.experimental.pallas import tpu_sc as plsc

B = 16384
H = 128
NC = 2
NS = 16
NW = NC * NS
BPW = B // NW
CHUNK = 128
NCHUNK = BPW // CHUNK
NVEC = H // 16

BM = 2048


def _gather_body(pt_hbm, ct_hbm, bt_hbm, ip_hbm, ic_hbm, ib_hbm,
                 out_hbm,
                 idxp, idxc, idxb, acc, buf, ctab_s, btab_s,
                 isem, psem, bsem0, bsem1, osem):
    wid = lax.axis_index("s") * NC + lax.axis_index("c")
    base = wid * BPW

    idescs = []
    for src, dst in ((ip_hbm, idxp), (ic_hbm, idxc), (ib_hbm, idxb)):
        for j in range(NCHUNK):
            idescs.append(pltpu.async_copy(
                src.at[pl.ds(base + j * CHUNK, CHUNK)], dst.at[j], isem))

    @pl.when(lax.axis_index("s") == 0)
    def _():
        pltpu.sync_copy(ct_hbm, ctab_s)
        pltpu.sync_copy(bt_hbm, btab_s)

    for d in idescs:
        d.wait()

    pdescs = [
        pltpu.async_copy(pt_hbm.at[idxp.at[j]],
                         acc.at[pl.ds(j * CHUNK, CHUNK)], psem)
        for j in range(NCHUNK)
    ]

    plsc.subcore_barrier()

    segs = []
    for j in range(NCHUNK):
        segs.append((ctab_s, idxc, j))
        segs.append((btab_s, idxb, j))
    bsems = (bsem0, bsem1)

    def fire(s):
        tab, idx, j = segs[s]
        k = s % 2
        return pltpu.async_copy(tab.at[idx.at[j]], buf.at[k], bsems[k])

    descs = {0: fire(0), 1: fire(1)}
    wdescs = []
    for s in range(len(segs)):
        k = s % 2
        j = segs[s][2]
        if s % 2 == 0:
            pdescs[j].wait()
        descs.pop(s).wait()
        cbase = j * CHUNK

        def add_body(i, _, k=k, cbase=cbase):
            r0 = i * 2
            r1 = r0 + 1
            for u in range(NVEC):
                c = u * 16
                plsc.addupdate(acc.at[cbase + r0, pl.ds(c, 16)],
                               buf[k, r0, pl.ds(c, 16)])
                plsc.addupdate(acc.at[cbase + r1, pl.ds(c, 16)],
                               buf[k, r1, pl.ds(c, 16)])
            return _

        lax.fori_loop(0, CHUNK // 2, add_body, None)
        if s + 2 < len(segs):
            descs[s + 2] = fire(s + 2)
        if s % 2 == 1:
            wdescs.append(pltpu.async_copy(
                acc.at[pl.ds(cbase, CHUNK)],
                out_hbm.at[pl.ds(base + cbase, CHUNK)], osem))

    for d in wdescs:
        d.wait()


_gather = pl.kernel(
    _gather_body,
    out_type=jax.ShapeDtypeStruct((B, H), jnp.float32),
    mesh=plsc.VectorSubcoreMesh(core_axis_name="c", subcore_axis_name="s",
                                num_cores=NC, num_subcores=NS),
    scratch_types=[
        pltpu.VMEM((NCHUNK, CHUNK), jnp.int32),
        pltpu.VMEM((NCHUNK, CHUNK), jnp.int32),
        pltpu.VMEM((NCHUNK, CHUNK), jnp.int32),
        pltpu.VMEM((BPW, H), jnp.float32),
        pltpu.VMEM((2, CHUNK, H), jnp.float32),
        pltpu.VMEM_SHARED((1000, H), jnp.float32),
        pltpu.VMEM_SHARED((1000, H), jnp.float32),
        pltpu.SemaphoreType.DMA,
        pltpu.SemaphoreType.DMA,
        pltpu.SemaphoreType.DMA,
        pltpu.SemaphoreType.DMA,
        pltpu.SemaphoreType.DMA,
    ],
)


def _mlp_body(h_ref, w1_ref, b1_ref, w2_ref, b2_ref, out_ref):
    h = h_ref[...]
    a = jnp.dot(h, w1_ref[...], preferred_element_type=jnp.float32) + b1_ref[...]
    a = a * jax.nn.sigmoid(a)
    out_ref[...] = (jnp.dot(a, w2_ref[...], preferred_element_type=jnp.float32)
                    + b2_ref[...])


def _mlp(hidden, W1, b1, W2, b2):
    grid = (B // BM,)
    row_spec = pl.BlockSpec((BM, H), lambda i: (i, 0))
    full = pl.BlockSpec((H, H), lambda i: (0, 0))
    bias = pl.BlockSpec((1, H), lambda i: (0, 0))
    return pl.pallas_call(
        _mlp_body,
        grid=grid,
        in_specs=[row_spec, full, bias, full, bias],
        out_specs=row_spec,
        out_shape=jax.ShapeDtypeStruct((B, H), jnp.float32),
    )(hidden, W1, b1.reshape(1, H), W2, b2.reshape(1, H))


def kernel(perturbation, cell_type, batch, perturb_table, cell_table,
           batch_table, W1, b1, W2, b2):
    ip = perturbation.astype(jnp.int32)
    ic = cell_type.astype(jnp.int32)
    ib = batch.astype(jnp.int32)
    hidden = _gather(perturb_table, cell_table, batch_table, ip, ic, ib)
    return _mlp(hidden, W1, b1, W2, b2)


# speedup vs baseline: 1.2762x; 1.0759x over previous
import jax
import jax.numpy as jnp
from jax import lax
from jax.experimental import pallas as pl
from jax.experimental.pallas import tpu as pltpu
from jax.experimental.pallas import tpu_sc as plsc

B = 16384
H = 128
NC = 2
NS = 16
NW = NC * NS
BPW = B // NW
CHUNK = 128
NCHUNK = BPW // CHUNK
NVEC = H // 16

BM = 2048


def _gather_body(pt_hbm, ct_hbm, bt_hbm, ip_hbm, ic_hbm, ib_hbm,
                 out_hbm,
                 idxp, idxc, idxb, acc, buf, ctab_s, btab_s,
                 isem, psem, bsem0, bsem1, osem):
    wid = lax.axis_index("s") * NC + lax.axis_index("c")
    base = wid * BPW

    idescs = []
    for src, dst in ((ip_hbm, idxp), (ic_hbm, idxc), (ib_hbm, idxb)):
        for j in range(NCHUNK):
            idescs.append(pltpu.async_copy(
                src.at[pl.ds(base + j * CHUNK, CHUNK)], dst.at[j], isem))

    @pl.when(lax.axis_index("s") == 0)
    def _():
        pltpu.sync_copy(ct_hbm, ctab_s)
        pltpu.sync_copy(bt_hbm, btab_s)

    for d in idescs:
        d.wait()

    pdescs = [
        pltpu.async_copy(pt_hbm.at[idxp.at[j]],
                         acc.at[pl.ds(j * CHUNK, CHUNK)], psem)
        for j in range(NCHUNK)
    ]

    plsc.subcore_barrier()

    cdescs = {}
    for j in range(NCHUNK):
        pdescs[j].wait()
        cdescs[j] = pltpu.async_copy(
            ctab_s.at[idxc.at[j]], acc.at[pl.ds(j * CHUNK, CHUNK)], bsem0,
            add=True)
    bdescs = {}
    for j in range(NCHUNK):
        cdescs[j].wait()
        bdescs[j] = pltpu.async_copy(
            btab_s.at[idxb.at[j]], acc.at[pl.ds(j * CHUNK, CHUNK)], bsem1,
            add=True)
    wdescs = []
    for j in range(NCHUNK):
        bdescs[j].wait()
        wdescs.append(pltpu.async_copy(
            acc.at[pl.ds(j * CHUNK, CHUNK)],
            out_hbm.at[pl.ds(base + j * CHUNK, CHUNK)], osem))
    for d in wdescs:
        d.wait()


_gather = pl.kernel(
    _gather_body,
    out_type=jax.ShapeDtypeStruct((B, H), jnp.float32),
    mesh=plsc.VectorSubcoreMesh(core_axis_name="c", subcore_axis_name="s",
                                num_cores=NC, num_subcores=NS),
    scratch_types=[
        pltpu.VMEM((NCHUNK, CHUNK), jnp.int32),
        pltpu.VMEM((NCHUNK, CHUNK), jnp.int32),
        pltpu.VMEM((NCHUNK, CHUNK), jnp.int32),
        pltpu.VMEM((BPW, H), jnp.float32),
        pltpu.VMEM((2, CHUNK, H), jnp.float32),
        pltpu.VMEM_SHARED((1000, H), jnp.float32),
        pltpu.VMEM_SHARED((1000, H), jnp.float32),
        pltpu.SemaphoreType.DMA,
        pltpu.SemaphoreType.DMA,
        pltpu.SemaphoreType.DMA,
        pltpu.SemaphoreType.DMA,
        pltpu.SemaphoreType.DMA,
    ],
)


def _mlp_body(h_ref, w1_ref, b1_ref, w2_ref, b2_ref, out_ref):
    h = h_ref[...]
    a = jnp.dot(h, w1_ref[...], preferred_element_type=jnp.float32) + b1_ref[...]
    a = a * jax.nn.sigmoid(a)
    out_ref[...] = (jnp.dot(a, w2_ref[...], preferred_element_type=jnp.float32)
                    + b2_ref[...])


def _mlp(hidden, W1, b1, W2, b2):
    grid = (B // BM,)
    row_spec = pl.BlockSpec((BM, H), lambda i: (i, 0))
    full = pl.BlockSpec((H, H), lambda i: (0, 0))
    bias = pl.BlockSpec((1, H), lambda i: (0, 0))
    return pl.pallas_call(
        _mlp_body,
        grid=grid,
        in_specs=[row_spec, full, bias, full, bias],
        out_specs=row_spec,
        out_shape=jax.ShapeDtypeStruct((B, H), jnp.float32),
    )(hidden, W1, b1.reshape(1, H), W2, b2.reshape(1, H))


def kernel(perturbation, cell_type, batch, perturb_table, cell_table,
           batch_table, W1, b1, W2, b2):
    ip = perturbation.astype(jnp.int32)
    ic = cell_type.astype(jnp.int32)
    ib = batch.astype(jnp.int32)
    hidden = _gather(perturb_table, cell_table, batch_table, ip, ic, ib)
    return _mlp(hidden, W1, b1, W2, b2)
